# baseline (device time: 27503 ns/iter reference)
import jax
import jax.numpy as jnp
from jax import lax
from jax.experimental import pallas as pl
from jax.experimental.pallas import tpu as pltpu

NC = 8


def kernel(partial, resid, gamma):
    m, d = partial.shape[-2], partial.shape[-1]
    half = m // 2
    quart = half // 2
    cr = quart // NC

    def body(p_hbm, r_hbm, g_ref, out_hbm,
             pn_v, pb_v, x_comm, p_v, r_v, out_v, outb, zin, xyin, ylin,
             oq2, oq3, oq4,
             pn_sem, p_sem, r_sem, wb1, wb2, wb3, wb4,
             rs_send, rs_recv, xag_send, xag_recv, yag_send, yag_recv,
             z_send, z_recv, yl_send, yl_recv):
        my_x = lax.axis_index("x")
        my_y = lax.axis_index("y")
        my_z = lax.axis_index("z")
        h = (my_x + my_y) % 2
        s = my_z % 2
        x_peer = (1 - my_x, my_y, my_z)
        y_peer = (my_x, my_y ^ 1, my_z)
        z_peer = (my_x, my_y, my_z ^ 1)

        barrier_sem = pltpu.get_barrier_semaphore()
        for nbr in (x_peer, y_peer, z_peer):
            pl.semaphore_signal(
                barrier_sem, inc=1, device_id=nbr,
                device_id_type=pl.DeviceIdType.MESH,
            )
        pl.semaphore_wait(barrier_sem, 3)

        q_mine = h * half + s * quart
        q_zq = h * half + (1 - s) * quart
        q_xe = (1 - h) * half + s * quart
        q_yl = (1 - h) * half + (1 - s) * quart

        cp_n = pltpu.make_async_copy(
            p_hbm.at[0, pl.ds(q_xe, quart), :], pn_v, pn_sem)
        cp_p = pltpu.make_async_copy(
            p_hbm.at[0, pl.ds(q_mine, quart), :], p_v, p_sem)
        cp_r = pltpu.make_async_copy(
            r_hbm.at[pl.ds(q_mine, quart), :], r_v, r_sem)
        cp_n.start()
        cp_p.start()
        cp_r.start()
        cp_n.wait()

        rs = []
        for j in range(NC):
            loc = pl.ds(j * cr, cr)
            pb_v[j] = pn_v[loc, :].astype(jnp.bfloat16)
            rdma = pltpu.make_async_remote_copy(
                src_ref=pb_v.at[j],
                dst_ref=x_comm.at[j],
                send_sem=rs_send.at[j],
                recv_sem=rs_recv.at[j],
                device_id=x_peer,
                device_id_type=pl.DeviceIdType.MESH,
            )
            rdma.start()
            rs.append(rdma)

        cp_p.wait()
        cp_r.wait()

        ag = []
        zs = []
        wbs = []
        for j in range(NC):
            rs[j].wait_recv()
            loc = pl.ds(j * cr, cr)
            glo = pl.ds(q_mine + j * cr, cr)
            y = p_v[loc, :] + x_comm[j].astype(jnp.float32) + r_v[loc, :]
            rms = jnp.sqrt(jnp.mean(y * y, axis=-1, keepdims=True) + 1e-6)
            res = y / rms * g_ref[...]
            out_v[loc, :] = res
            outb[j] = res.astype(jnp.bfloat16)
            zr = pltpu.make_async_remote_copy(
                src_ref=outb.at[j],
                dst_ref=zin.at[j],
                send_sem=z_send.at[j],
                recv_sem=z_recv.at[j],
                device_id=z_peer,
                device_id_type=pl.DeviceIdType.MESH,
            )
            zr.start()
            zs.append(zr)
            if j < NC // 2:
                peer, ssem, rsem = x_peer, xag_send.at[j], xag_recv.at[j]
            else:
                peer, ssem, rsem = (
                    y_peer, yag_send.at[j - NC // 2], yag_recv.at[j - NC // 2])
            rdma = pltpu.make_async_remote_copy(
                src_ref=outb.at[j],
                dst_ref=xyin.at[j],
                send_sem=ssem,
                recv_sem=rsem,
                device_id=peer,
                device_id_type=pl.DeviceIdType.MESH,
            )
            rdma.start()
            ag.append(rdma)
            wb = pltpu.make_async_copy(
                out_v.at[loc, :], out_hbm.at[glo, :], wb1.at[j])
            wb.start()
            wbs.append(wb)

        yl = []
        for j in range(NC):
            zs[j].wait_recv()
            loc = pl.ds(j * cr, cr)
            rdma = pltpu.make_async_remote_copy(
                src_ref=zin.at[j],
                dst_ref=ylin.at[j],
                send_sem=yl_send.at[j],
                recv_sem=yl_recv.at[j],
                device_id=y_peer,
                device_id_type=pl.DeviceIdType.MESH,
            )
            rdma.start()
            yl.append(rdma)
            oq2[loc, :] = zin[j].astype(jnp.float32)
            wb = pltpu.make_async_copy(
                oq2.at[loc, :], out_hbm.at[pl.ds(q_zq + j * cr, cr), :],
                wb2.at[j])
            wb.start()
            wbs.append(wb)

        for j in range(NC):
            ag[j].wait_recv()
            loc = pl.ds(j * cr, cr)
            oq3[loc, :] = xyin[j].astype(jnp.float32)
            wb = pltpu.make_async_copy(
                oq3.at[loc, :], out_hbm.at[pl.ds(q_xe + j * cr, cr), :],
                wb3.at[j])
            wb.start()
            wbs.append(wb)
        for j in range(NC):
            yl[j].wait_recv()
            loc = pl.ds(j * cr, cr)
            oq4[loc, :] = ylin[j].astype(jnp.float32)
            wb = pltpu.make_async_copy(
                oq4.at[loc, :], out_hbm.at[pl.ds(q_yl + j * cr, cr), :],
                wb4.at[j])
            wb.start()
            wbs.append(wb)

        for j in range(NC):
            rs[j].wait_send()
            ag[j].wait_send()
            zs[j].wait_send()
            yl[j].wait_send()
        for w in wbs:
            w.wait()

    return pl.pallas_call(
        body,
        out_shape=jax.ShapeDtypeStruct((m, d), jnp.float32),
        in_specs=[
            pl.BlockSpec(memory_space=pltpu.MemorySpace.HBM),
            pl.BlockSpec(memory_space=pltpu.MemorySpace.HBM),
            pl.BlockSpec(memory_space=pltpu.VMEM),
        ],
        out_specs=pl.BlockSpec(memory_space=pltpu.MemorySpace.HBM),
        scratch_shapes=[
            pltpu.VMEM((quart, d), jnp.float32),
            pltpu.VMEM((NC, cr, d), jnp.bfloat16),
            pltpu.VMEM((NC, cr, d), jnp.bfloat16),
            pltpu.VMEM((quart, d), jnp.float32),
            pltpu.VMEM((quart, d), jnp.float32),
            pltpu.VMEM((quart, d), jnp.float32),
            pltpu.VMEM((NC, cr, d), jnp.bfloat16),
            pltpu.VMEM((NC, cr, d), jnp.bfloat16),
            pltpu.VMEM((NC, cr, d), jnp.bfloat16),
            pltpu.VMEM((NC, cr, d), jnp.bfloat16),
            pltpu.VMEM((quart, d), jnp.float32),
            pltpu.VMEM((quart, d), jnp.float32),
            pltpu.VMEM((quart, d), jnp.float32),
            pltpu.SemaphoreType.DMA,
            pltpu.SemaphoreType.DMA,
            pltpu.SemaphoreType.DMA,
            pltpu.SemaphoreType.DMA((NC,)),
            pltpu.SemaphoreType.DMA((NC,)),
            pltpu.SemaphoreType.DMA((NC,)),
            pltpu.SemaphoreType.DMA((NC,)),
            pltpu.SemaphoreType.DMA((NC,)),
            pltpu.SemaphoreType.DMA((NC,)),
            pltpu.SemaphoreType.DMA((NC // 2,)),
            pltpu.SemaphoreType.DMA((NC // 2,)),
            pltpu.SemaphoreType.DMA((NC // 2,)),
            pltpu.SemaphoreType.DMA((NC // 2,)),
            pltpu.SemaphoreType.DMA((NC,)),
            pltpu.SemaphoreType.DMA((NC,)),
            pltpu.SemaphoreType.DMA((NC,)),
            pltpu.SemaphoreType.DMA((NC,)),
        ],
        compiler_params=pltpu.CompilerParams(collective_id=0),
    )(partial, resid, gamma)
